# baseline (speedup 1.0000x reference)
import numpy as np
import jax
import jax.numpy as jnp
from jax import lax
from jax.experimental import pallas as pl
from jax.experimental.pallas import tpu as pltpu
from jax.experimental.pallas import tpu_sc as plsc

B = 4
N = 150000
C = 5
SPAN = 9472
NPAD = SPAN * 16
ACH = 592
NVC = ACH // 16
NCH = NPAD // ACH
PSTART = 149776
PTAIL = N - PSTART
NX = 512
NBINS = NX * NX
BPW = NBINS // 16
BIG = NBINS
MAXV = 30000
MAXP = 20
QUART = 7504
QLAST = MAXV - 3 * QUART
NQ = 4
COLS = B * MAXV
PLANE = COLS
NPLANES = C * MAXP
FEAT = NPLANES * PLANE
FCELLS = NPLANES * QUART
FDUMP = FCELLS
FBLK = 750592
FSH = FBLK // 16
CBLK = MAXV * 4
CDUMP = CBLK
QSTR = 164096
QCH = 640
SENT = MAXP * 32768

_NORM_RANGE = np.array([-51.2, -51.2, -5.0, 0.0, 51.2, 51.2, 3.0, 255.0],
                       dtype=np.float32)
_STARTS = [float(_NORM_RANGE[i]) for i in range(4)]
_NORMS = [float(_NORM_RANGE[i + 4] - _NORM_RANGE[i]) for i in range(4)]

CSH_A = 7504
CSH_B = CBLK - 15 * CSH_A
ZB = 1024


def _loop(n, body):
  def f(i, c):
    body(i)
    return c
  lax.fori_loop(0, n, f, jnp.int32(0))


def _sc_body(pts, feat, coors, lin_hbm, qarr,
             pbuf, linbuf, count, slot, zbuf, cpat,
             si0, si1, si2, si3, si4, sv0, sv1, sv2, sv3, sv4,
             cidx1, cidx2, cidx3, cvz, cvy, cvx, sq,
             fblock, cblock, exch, exchv, sem):
  cid = lax.axis_index("c")
  sid = lax.axis_index("s")
  iota = lax.iota(jnp.int32, 16)
  ones_i = jnp.ones((16,), jnp.int32)
  zero_i = jnp.zeros((16,), jnp.int32)
  zero_f = jnp.zeros((16,), jnp.float32)
  sidx = [si0, si1, si2, si3, si4]
  sval = [sv0, sv1, sv2, sv3, sv4]
  cols = [jnp.full((16,), c, jnp.int32) for c in range(5)]
  lo = sid * BPW

  def zb(i):
    zbuf[pl.ds(i * 16, 16)] = zero_f
  _loop(ZB // 16, zb)

  def prefill_feat_staging():
    fdump = jnp.full((16,), FDUMP, jnp.int32)
    def pf(i):
      r = i // 8
      cl = (i % 8) * 16
      for s in sidx:
        s[r, pl.ds(cl, 16)] = fdump
    _loop(10 * 8, pf)

  def prefill_coors_staging():
    cdump = jnp.full((16,), CDUMP, jnp.int32)
    def pc(i):
      r = i // 8
      cl = (i % 8) * 16
      cidx1[r, pl.ds(cl, 16)] = cdump
      cidx2[r, pl.ds(cl, 16)] = cdump
      cidx3[r, pl.ds(cl, 16)] = cdump
      cvz[r, pl.ds(cl, 16)] = zero_i
    _loop(4 * 8, pc)

  def points_chunk_in(b, start, par):
    @pl.when(start + ACH <= N)
    def _():
      pltpu.sync_copy(pts.at[b, pl.ds(start, ACH), :], pbuf.at[par])

    @pl.when(start == PSTART)
    def _():
      pltpu.sync_copy(pts.at[b, pl.ds(PSTART, PTAIL), :],
                      pbuf.at[par].at[pl.ds(0, PTAIL), :])

  def chunk_fire(b, start, par):
    pltpu.async_copy(lin_hbm.at[pl.ds(b * NPAD + start, ACH)],
                     linbuf.at[par], sem)

    @pl.when(start + ACH <= N)
    def _():
      pltpu.async_copy(pts.at[b, pl.ds(start, ACH), :], pbuf.at[par], sem)

    @pl.when(start == PSTART)
    def _():
      pltpu.async_copy(pts.at[b, pl.ds(PSTART, PTAIL), :],
                       pbuf.at[par].at[pl.ds(0, PTAIL), :], sem)

  def chunk_wait(b, start, par):
    pltpu.make_async_copy(lin_hbm.at[pl.ds(b * NPAD + start, ACH)],
                          linbuf.at[par], sem).wait()

    @pl.when(start + ACH <= N)
    def _():
      pltpu.make_async_copy(pts.at[b, pl.ds(start, ACH), :], pbuf.at[par],
                            sem).wait()

    @pl.when(start == PSTART)
    def _():
      pltpu.make_async_copy(pts.at[b, pl.ds(PSTART, PTAIL), :],
                            pbuf.at[par].at[pl.ds(0, PTAIL), :], sem).wait()

  def phase_a(b):
    base_pt = sid * SPAN

    zv = jnp.zeros((16,), jnp.int32)

    def chunk_body(ci, _):
      start = base_pt + ci * ACH
      points_chunk_in(b, start, 0)

      def vec(j, _):
        rows = j * 16 + iota
        gi = start + rows
        x = plsc.load_gather(pbuf, [zv, rows, cols[0]])
        y = plsc.load_gather(pbuf, [zv, rows, cols[1]])
        z = plsc.load_gather(pbuf, [zv, rows, cols[2]])
        tx = (x - jnp.float32(-51.2)) / jnp.float32(0.2)
        ty = (y - jnp.float32(-51.2)) / jnp.float32(0.2)
        tz = (z - jnp.float32(-5.0)) / jnp.float32(8.0)
        ok = ((tx >= 0.0) & (tx < 512.0)
              & (ty >= 0.0) & (ty < 512.0)
              & (tz >= 0.0) & (tz < 1.0)
              & (gi < N))
        xi = jnp.clip(tx, 0.0, 513.0).astype(jnp.int32)
        yi = jnp.clip(ty, 0.0, 513.0).astype(jnp.int32)
        l = jnp.where(ok, yi * NX + xi, BIG)
        linbuf[0, pl.ds(j * 16, 16)] = l
        return jnp.int32(0)

      lax.fori_loop(0, NVC, vec, jnp.int32(0))
      pltpu.sync_copy(linbuf.at[0], lin_hbm.at[pl.ds(b * NPAD + start, ACH)])
      return jnp.int32(0)

    lax.fori_loop(0, SPAN // ACH, chunk_body, jnp.int32(0))

  def init_cblock(b):
    cpvec = jnp.where(iota % 4 == 0, jnp.full((16,), b, jnp.int32),
                      jnp.full((16,), -1, jnp.int32))

    def cp(i):
      cpat[pl.ds(i * 16, 16)] = cpvec
    _loop(960 // 16, cp)

    el0 = sid * CSH_A
    ds = []
    for q in range(7):
      ds.append(pltpu.async_copy(cpat, cblock.at[pl.ds(el0 + q * 960, 960)],
                                 sem))

    @pl.when(sid < 15)
    def _():
      d = pltpu.async_copy(cpat.at[pl.ds(0, CSH_A - 6720)],
                           cblock.at[pl.ds(el0 + 6720, CSH_A - 6720)], sem)
      d.wait()

    @pl.when(sid == 15)
    def _():
      d = pltpu.async_copy(cpat.at[pl.ds(0, CSH_B - 6720)],
                           cblock.at[pl.ds(el0 + 6720, CSH_B - 6720)], sem)
      d.wait()
    for d in ds:
      d.wait()

  def b1_count(b):
    def z(i):
      count[pl.ds(i * 16, 16)] = zero_i
    _loop(BPW // 16, z)

    def chunk(ci, _):
      pltpu.sync_copy(lin_hbm.at[pl.ds(b * NPAD + ci * ACH, ACH)],
                      linbuf.at[0])

      def vec(j, _):
        v = linbuf[0, pl.ds(j * 16, 16)]
        m = (v >= lo) & (v < lo + BPW)
        locc = jnp.where(m, v - lo, 0)
        plsc.addupdate_scatter(count, [locc], ones_i, mask=m)
        return jnp.int32(0)

      lax.fori_loop(0, NVC, vec, jnp.int32(0))
      return jnp.int32(0)

    lax.fori_loop(0, NCH, chunk, jnp.int32(0))

  def prefix_and_base():
    def pj(j, carry):
      c16 = count[pl.ds(j * 16, 16)]
      occ = (c16 > 0).astype(jnp.int32)
      s16 = plsc.cumsum(occ)
      slot[pl.ds(j * 16, 16)] = (carry + s16 - occ) * 2 + occ
      return carry + jnp.sum(occ)

    d = lax.fori_loop(0, BPW // 16, pj, jnp.int32(0))
    linbuf[0, pl.ds(0, 16)] = jnp.full((16,), d, jnp.int32)
    pltpu.sync_copy(linbuf.at[0].at[pl.ds(0, 16)], exch.at[sid])
    plsc.subcore_barrier()
    pltpu.sync_copy(exch, exchv)
    allv = plsc.load_gather(exchv, [iota, iota])
    base = jnp.sum(jnp.where(iota < sid, allv, 0))
    return base

  def coors_scatter(b, base):
    prefill_coors_staging()

    def ch(ci, _):
      choff = ci * 512

      def vec(j, noff):
        off16 = choff + j * 16
        enc = slot[pl.ds(off16, 16)]
        slv = (enc >> 1) + base
        occm = ((enc & 1) > 0) & (slv < MAXV)

        def proc(noff):
          binv = lo + off16 + iota
          yv = binv >> 9
          xv = binv & (NX - 1)
          om = occm.astype(jnp.int32)
          pos = noff + plsc.cumsum(om) - 1
          ph = pos >> 7
          pcl = pos & 127
          r4 = slv * 4
          plsc.store_scatter(cidx1, [ph, pcl], r4 + 1, mask=occm)
          plsc.store_scatter(cidx2, [ph, pcl], r4 + 2, mask=occm)
          plsc.store_scatter(cidx3, [ph, pcl], r4 + 3, mask=occm)
          plsc.store_scatter(cvy, [ph, pcl], yv, mask=occm)
          plsc.store_scatter(cvx, [ph, pcl], xv, mask=occm)
          return noff + jnp.sum(om)

        return lax.cond(jnp.any(occm), proc, lambda n: n, noff)

      noff = lax.fori_loop(0, 32, vec, jnp.int32(0))
      nblk = (noff + 127) >> 7

      def fl(j, _):
        pltpu.async_copy(cvz.at[j], cblock.at[cidx1.at[j]], sem).wait()
        pltpu.async_copy(cvy.at[j], cblock.at[cidx2.at[j]], sem).wait()
        pltpu.async_copy(cvx.at[j], cblock.at[cidx3.at[j]], sem).wait()
        return jnp.int32(0)

      lax.fori_loop(0, nblk, fl, jnp.int32(0))
      return jnp.int32(0)

    lax.fori_loop(0, BPW // 512, ch, jnp.int32(0))
    plsc.subcore_barrier()
    el0 = sid * CSH_A

    @pl.when(sid < 15)
    def _():
      pltpu.sync_copy(cblock.at[pl.ds(el0, CSH_A)],
                      coors.at[pl.ds(b * CBLK + el0, CSH_A)])

    @pl.when(sid == 15)
    def _():
      pltpu.sync_copy(cblock.at[pl.ds(el0, CSH_B)],
                      coors.at[pl.ds(b * CBLK + el0, CSH_B)])
    plsc.subcore_barrier()

  def scan_queue(b, base, qbase):
    def zr(i):
      count[pl.ds(i * 16, 16)] = zero_i
    _loop(BPW // 16, zr)

    chunk_fire(b, 0, 0)

    def ch(ci, qnb):
      start = ci * ACH
      par = ci & 1
      chunk_wait(b, start, par)

      @pl.when(ci + 1 < NCH)
      def _():
        chunk_fire(b, start + ACH, 1 - par)

      parv = jnp.full((16,), par, jnp.int32)

      def vec(j, off):
        v = linbuf[par, pl.ds(j * 16, 16)]
        m = (v >= lo) & (v < lo + BPW)

        def proc(off):
          locc = jnp.where(m, v - lo, 0)
          cnt = plsc.load_gather(count, [locc], mask=m)
          rc, _lm = plsc.scan_count(locc, m)
          plsc.addupdate_scatter(count, [locc], ones_i, mask=m)
          rank = cnt + rc - 1
          enc = plsc.load_gather(slot, [locc], mask=m)
          slv = (enc >> 1) + base
          valid = m & (rank < MAXP) & (slv < MAXV)
          code = rank * 32768 + slv
          vi = valid.astype(jnp.int32)
          pos = off + plsc.cumsum(vi) - 1
          spos = pos * 6
          rows = j * 16 + iota
          plsc.store_scatter(sq, [spos], code, mask=valid)
          for c in range(5):
            val = plsc.load_gather(pbuf, [parv, rows, cols[c]], mask=valid)
            if c < 4:
              val = (val - jnp.float32(_STARTS[c])) / jnp.float32(_NORMS[c])
            plsc.store_scatter(sq, [spos + 1 + c],
                               plsc.bitcast(val, jnp.int32), mask=valid)
          return off + jnp.sum(vi)

        return lax.cond(jnp.any(m), proc, lambda o: o, off)

      off = lax.fori_loop(0, NVC, vec, jnp.int32(0))
      nblk = (off + 127) >> 7

      def pad(i):
        pp = off + i * 16 + iota
        mk = pp < nblk * 128
        plsc.store_scatter(sq, [pp * 6], jnp.full((16,), SENT, jnp.int32),
                           mask=mk)
      _loop(8, pad)

      def fl(j, _):
        pltpu.sync_copy(sq.at[pl.ds(j * 768, 768)],
                        qarr.at[pl.ds(qbase + (qnb + j) * 768, 768)])
        return jnp.int32(0)

      lax.fori_loop(0, nblk, fl, jnp.int32(0))
      return qnb + nblk

    return lax.fori_loop(0, NCH, ch, jnp.int32(0))

  def quarter_pass(b, qnb, qbase, h):
    qn = qnb * 128
    bcol = b * MAXV
    slo = h * QUART
    nfz = FSH // ZB
    rem = FSH - nfz * ZB

    def zf(i, _):
      pltpu.async_copy(zbuf, fblock.at[pl.ds(sid * FSH + i * ZB, ZB)], sem
                       ).wait()
      return jnp.int32(0)

    lax.fori_loop(0, nfz, zf, jnp.int32(0))
    pltpu.async_copy(zbuf.at[pl.ds(0, rem)],
                     fblock.at[pl.ds(sid * FSH + nfz * ZB, rem)], sem).wait()
    prefill_feat_staging()
    plsc.subcore_barrier()

    nqc = (qn + QCH - 1) // QCH

    def ch(k, _):
      pltpu.sync_copy(qarr.at[pl.ds(qbase + k * (QCH * 6), QCH * 6)], sq)

      def vec(j, off):
        e = j * 16 + iota
        ge = k * QCH + e
        e6 = e * 6
        code = plsc.load_gather(sq, [e6])
        rank = code >> 15
        slv = code & 32767
        sloc = slv - slo
        validq = ((ge < qn) & (rank < MAXP) & (slv < MAXV)
                  & (sloc >= 0) & (sloc < QUART))

        def proc(off):
          d0 = rank * QUART + sloc
          vi = validq.astype(jnp.int32)
          pos = off + plsc.cumsum(vi) - 1
          ph = pos >> 7
          pcl = pos & 127
          for c in range(5):
            dst = d0 + c * (MAXP * QUART)
            plsc.store_scatter(sidx[c], [ph, pcl], dst, mask=validq)
            vv = plsc.load_gather(sq, [e6 + 1 + c], mask=validq)
            plsc.store_scatter(sval[c], [ph, pcl],
                               plsc.bitcast(vv, jnp.float32), mask=validq)
          return off + jnp.sum(vi)

        return lax.cond(jnp.any(validq), proc, lambda o: o, off)

      off = lax.fori_loop(0, QCH // 16, vec, jnp.int32(0))
      nblk = (off + 127) >> 7

      def fl(j, _):
        for c in range(5):
          pltpu.async_copy(sval[c].at[j], fblock.at[sidx[c].at[j]], sem
                           ).wait()
        return jnp.int32(0)

      lax.fori_loop(0, nblk, fl, jnp.int32(0))
      return jnp.int32(0)

    lax.fori_loop(0, nqc, ch, jnp.int32(0))
    plsc.subcore_barrier()
    for kk in range(7):
      p = sid + 16 * kk

      @pl.when((p < NPLANES) & (h < 3))
      def _():
        pltpu.async_copy(fblock.at[pl.ds(p * QUART, QUART)],
                         feat.at[pl.ds(p * PLANE + bcol + slo, QUART)],
                         sem).wait()

      @pl.when((p < NPLANES) & (h == 3))
      def _():
        pltpu.async_copy(fblock.at[pl.ds(p * QUART, QLAST)],
                         feat.at[pl.ds(p * PLANE + bcol + slo, QLAST)],
                         sem).wait()
    plsc.subcore_barrier()

  def batch_body(k, _):
    b = 2 * cid + k
    with jax.named_scope("phase_a"):
      phase_a(b)
    with jax.named_scope("init_cblock"):
      init_cblock(b)
    plsc.subcore_barrier()
    with jax.named_scope("b1_count"):
      b1_count(b)
    with jax.named_scope("prefix"):
      base = prefix_and_base()
    with jax.named_scope("coors_scatter"):
      coors_scatter(b, base)
    qbase = (cid * 16 + sid) * (QSTR * 6)
    with jax.named_scope("scan_queue"):
      qnb = scan_queue(b, base, qbase)

    def qloop(h, _):
      with jax.named_scope("quarter_pass"):
        quarter_pass(b, qnb, qbase, h)
      return jnp.int32(0)

    lax.fori_loop(0, NQ, qloop, jnp.int32(0))
    return jnp.int32(0)

  lax.fori_loop(0, 2, batch_body, jnp.int32(0))


def kernel(points_lst):
  mesh = plsc.VectorSubcoreMesh(core_axis_name="c", subcore_axis_name="s")
  kfn = pl.kernel(
      _sc_body,
      out_type=(
          jax.ShapeDtypeStruct((FEAT,), jnp.float32),
          jax.ShapeDtypeStruct((COLS * 4,), jnp.int32),
          jax.ShapeDtypeStruct((B * NPAD,), jnp.int32),
          jax.ShapeDtypeStruct((32 * QSTR * 6,), jnp.int32),
      ),
      mesh=mesh,
      scratch_types=[
          pltpu.VMEM((2, ACH, C), jnp.float32),
          pltpu.VMEM((2, ACH), jnp.int32),
          pltpu.VMEM((BPW,), jnp.int32),
          pltpu.VMEM((BPW,), jnp.int32),
          pltpu.VMEM((ZB,), jnp.float32),
          pltpu.VMEM((960,), jnp.int32),
          pltpu.VMEM((10, 128), jnp.int32),
          pltpu.VMEM((10, 128), jnp.int32),
          pltpu.VMEM((10, 128), jnp.int32),
          pltpu.VMEM((10, 128), jnp.int32),
          pltpu.VMEM((10, 128), jnp.int32),
          pltpu.VMEM((10, 128), jnp.float32),
          pltpu.VMEM((10, 128), jnp.float32),
          pltpu.VMEM((10, 128), jnp.float32),
          pltpu.VMEM((10, 128), jnp.float32),
          pltpu.VMEM((10, 128), jnp.float32),
          pltpu.VMEM((4, 128), jnp.int32),
          pltpu.VMEM((4, 128), jnp.int32),
          pltpu.VMEM((4, 128), jnp.int32),
          pltpu.VMEM((4, 128), jnp.int32),
          pltpu.VMEM((4, 128), jnp.int32),
          pltpu.VMEM((4, 128), jnp.int32),
          pltpu.VMEM((3840,), jnp.int32),
          pltpu.VMEM_SHARED((FBLK,), jnp.float32),
          pltpu.VMEM_SHARED((CBLK + 64,), jnp.int32),
          pltpu.VMEM_SHARED((16, 16), jnp.int32),
          pltpu.VMEM((16, 16), jnp.int32),
          pltpu.SemaphoreType.DMA,
      ],
      compiler_params=pltpu.CompilerParams(
          needs_layout_passes=False, use_tc_tiling_on_sc=False),
  )
  feat, coors, _lin, _q = kfn(points_lst)
  features = feat.reshape(1, C, MAXP, COLS)
  coors_batch = coors.reshape(COLS, 4)
  return features, coors_batch


# speedup vs baseline: 1.2219x; 1.2219x over previous
import numpy as np
import jax
import jax.numpy as jnp
from jax import lax
from jax.experimental import pallas as pl
from jax.experimental.pallas import tpu as pltpu
from jax.experimental.pallas import tpu_sc as plsc

B = 4
N = 150000
C = 5
SPAN = 9472
NPAD = SPAN * 16
ACH = 1184
NVC = ACH // 16
NCH = NPAD // ACH
PSTART = 149184
PTAIL = N - PSTART
NX = 512
NBINS = NX * NX
BPW = NBINS // 16
BIG = NBINS
MAXV = 30000
MAXP = 20
QUART = 7504
QLAST = MAXV - 3 * QUART
NQ = 4
COLS = B * MAXV
PLANE = COLS
NPLANES = C * MAXP
FEAT = NPLANES * PLANE
FCELLS = NPLANES * QUART
FDUMP = FCELLS
FBLK = 750592
FSH = FBLK // 16
CBLK = MAXV * 4
CDUMP = CBLK
QSTR = 164096
QCH = 640
SENT = MAXP * 32768

_NORM_RANGE = np.array([-51.2, -51.2, -5.0, 0.0, 51.2, 51.2, 3.0, 255.0],
                       dtype=np.float32)
_STARTS = [float(_NORM_RANGE[i]) for i in range(4)]
_NORMS = [float(_NORM_RANGE[i + 4] - _NORM_RANGE[i]) for i in range(4)]

CSH_A = 7504
CSH_B = CBLK - 15 * CSH_A
ZB = 1024


def _loop(n, body):
  def f(i, c):
    body(i)
    return c
  lax.fori_loop(0, n, f, jnp.int32(0))


def _sc_body(pts, feat, coors, lin_hbm, qarr,
             pbuf, linbuf, count, slot, zbuf, cpat,
             si0, si1, si2, si3, si4, sv0, sv1, sv2, sv3, sv4,
             cidx1, cidx2, cidx3, cvz, cvy, cvx, sq,
             fblock, cblock, exch, exchv, sem):
  cid = lax.axis_index("c")
  sid = lax.axis_index("s")
  iota = lax.iota(jnp.int32, 16)
  ones_i = jnp.ones((16,), jnp.int32)
  zero_i = jnp.zeros((16,), jnp.int32)
  zero_f = jnp.zeros((16,), jnp.float32)
  sidx = [si0, si1, si2, si3, si4]
  sval = [sv0, sv1, sv2, sv3, sv4]
  cols = [jnp.full((16,), c, jnp.int32) for c in range(5)]
  lo = sid * BPW

  def zb(i):
    zbuf[pl.ds(i * 16, 16)] = zero_f
  _loop(ZB // 16, zb)

  def prefill_feat_staging():
    fdump = jnp.full((16,), FDUMP, jnp.int32)
    def pf(i):
      r = i // 8
      cl = (i % 8) * 16
      for s in sidx:
        s[r, pl.ds(cl, 16)] = fdump
    _loop(5 * 8, pf)

  def prefill_coors_staging():
    cdump = jnp.full((16,), CDUMP, jnp.int32)
    def pc(i):
      r = i // 8
      cl = (i % 8) * 16
      cidx1[r, pl.ds(cl, 16)] = cdump
      cidx2[r, pl.ds(cl, 16)] = cdump
      cidx3[r, pl.ds(cl, 16)] = cdump
      cvz[r, pl.ds(cl, 16)] = zero_i
    _loop(4 * 8, pc)

  def points_chunk_in(b, start, par):
    @pl.when(start + ACH <= N)
    def _():
      pltpu.sync_copy(pts.at[b, pl.ds(start, ACH), :], pbuf.at[par])

    @pl.when(start == PSTART)
    def _():
      pltpu.sync_copy(pts.at[b, pl.ds(PSTART, PTAIL), :],
                      pbuf.at[par].at[pl.ds(0, PTAIL), :])

  def chunk_fire(b, start, par):
    pltpu.async_copy(lin_hbm.at[pl.ds(b * NPAD + start, ACH)],
                     linbuf.at[par], sem)

    @pl.when(start + ACH <= N)
    def _():
      pltpu.async_copy(pts.at[b, pl.ds(start, ACH), :], pbuf.at[par], sem)

    @pl.when(start == PSTART)
    def _():
      pltpu.async_copy(pts.at[b, pl.ds(PSTART, PTAIL), :],
                       pbuf.at[par].at[pl.ds(0, PTAIL), :], sem)

  def chunk_wait(b, start, par):
    pltpu.make_async_copy(lin_hbm.at[pl.ds(b * NPAD + start, ACH)],
                          linbuf.at[par], sem).wait()

    @pl.when(start + ACH <= N)
    def _():
      pltpu.make_async_copy(pts.at[b, pl.ds(start, ACH), :], pbuf.at[par],
                            sem).wait()

    @pl.when(start == PSTART)
    def _():
      pltpu.make_async_copy(pts.at[b, pl.ds(PSTART, PTAIL), :],
                            pbuf.at[par].at[pl.ds(0, PTAIL), :], sem).wait()

  def phase_a(b):
    base_pt = sid * SPAN

    zv = jnp.zeros((16,), jnp.int32)

    def chunk_body(ci, _):
      start = base_pt + ci * ACH
      points_chunk_in(b, start, 0)

      def vec(j, _):
        rows = j * 16 + iota
        gi = start + rows
        x = plsc.load_gather(pbuf, [zv, rows, cols[0]])
        y = plsc.load_gather(pbuf, [zv, rows, cols[1]])
        z = plsc.load_gather(pbuf, [zv, rows, cols[2]])
        tx = (x - jnp.float32(-51.2)) / jnp.float32(0.2)
        ty = (y - jnp.float32(-51.2)) / jnp.float32(0.2)
        tz = (z - jnp.float32(-5.0)) / jnp.float32(8.0)
        ok = ((tx >= 0.0) & (tx < 512.0)
              & (ty >= 0.0) & (ty < 512.0)
              & (tz >= 0.0) & (tz < 1.0)
              & (gi < N))
        xi = jnp.clip(tx, 0.0, 513.0).astype(jnp.int32)
        yi = jnp.clip(ty, 0.0, 513.0).astype(jnp.int32)
        l = jnp.where(ok, yi * NX + xi, BIG)
        linbuf[0, pl.ds(j * 16, 16)] = l
        return jnp.int32(0)

      lax.fori_loop(0, NVC, vec, jnp.int32(0))
      pltpu.sync_copy(linbuf.at[0], lin_hbm.at[pl.ds(b * NPAD + start, ACH)])
      return jnp.int32(0)

    lax.fori_loop(0, SPAN // ACH, chunk_body, jnp.int32(0))

  def init_cblock(b):
    cpvec = jnp.where(iota % 4 == 0, jnp.full((16,), b, jnp.int32),
                      jnp.full((16,), -1, jnp.int32))

    def cp(i):
      cpat[pl.ds(i * 16, 16)] = cpvec
    _loop(960 // 16, cp)

    el0 = sid * CSH_A
    ds = []
    for q in range(7):
      ds.append(pltpu.async_copy(cpat, cblock.at[pl.ds(el0 + q * 960, 960)],
                                 sem))

    @pl.when(sid < 15)
    def _():
      d = pltpu.async_copy(cpat.at[pl.ds(0, CSH_A - 6720)],
                           cblock.at[pl.ds(el0 + 6720, CSH_A - 6720)], sem)
      d.wait()

    @pl.when(sid == 15)
    def _():
      d = pltpu.async_copy(cpat.at[pl.ds(0, CSH_B - 6720)],
                           cblock.at[pl.ds(el0 + 6720, CSH_B - 6720)], sem)
      d.wait()
    for d in ds:
      d.wait()

  def b1_count(b):
    def z(i):
      count[pl.ds(i * 16, 16)] = zero_i
    _loop(BPW // 16, z)

    def chunk(ci, _):
      pltpu.sync_copy(lin_hbm.at[pl.ds(b * NPAD + ci * ACH, ACH)],
                      linbuf.at[0])

      def vec(j, _):
        v = linbuf[0, pl.ds(j * 16, 16)]
        m = (v >= lo) & (v < lo + BPW)
        locc = jnp.where(m, v - lo, 0)
        plsc.addupdate_scatter(count, [locc], ones_i, mask=m)
        return jnp.int32(0)

      lax.fori_loop(0, NVC, vec, jnp.int32(0))
      return jnp.int32(0)

    lax.fori_loop(0, NCH, chunk, jnp.int32(0))

  def prefix_and_base():
    def pj(j, carry):
      c16 = count[pl.ds(j * 16, 16)]
      occ = (c16 > 0).astype(jnp.int32)
      s16 = plsc.cumsum(occ)
      slot[pl.ds(j * 16, 16)] = (carry + s16 - occ) * 2 + occ
      return carry + jnp.sum(occ)

    d = lax.fori_loop(0, BPW // 16, pj, jnp.int32(0))
    linbuf[0, pl.ds(0, 16)] = jnp.full((16,), d, jnp.int32)
    pltpu.sync_copy(linbuf.at[0].at[pl.ds(0, 16)], exch.at[sid])
    plsc.subcore_barrier()
    pltpu.sync_copy(exch, exchv)
    allv = plsc.load_gather(exchv, [iota, iota])
    base = jnp.sum(jnp.where(iota < sid, allv, 0))
    return base

  def coors_scatter(b, base):
    prefill_coors_staging()

    def ch(ci, _):
      choff = ci * 512

      def vec(j, noff):
        off16 = choff + j * 16
        enc = slot[pl.ds(off16, 16)]
        slv = (enc >> 1) + base
        occm = ((enc & 1) > 0) & (slv < MAXV)

        def proc(noff):
          binv = lo + off16 + iota
          yv = binv >> 9
          xv = binv & (NX - 1)
          om = occm.astype(jnp.int32)
          pos = noff + plsc.cumsum(om) - 1
          ph = pos >> 7
          pcl = pos & 127
          r4 = slv * 4
          plsc.store_scatter(cidx1, [ph, pcl], r4 + 1, mask=occm)
          plsc.store_scatter(cidx2, [ph, pcl], r4 + 2, mask=occm)
          plsc.store_scatter(cidx3, [ph, pcl], r4 + 3, mask=occm)
          plsc.store_scatter(cvy, [ph, pcl], yv, mask=occm)
          plsc.store_scatter(cvx, [ph, pcl], xv, mask=occm)
          return noff + jnp.sum(om)

        return lax.cond(jnp.any(occm), proc, lambda n: n, noff)

      noff = lax.fori_loop(0, 32, vec, jnp.int32(0))
      nblk = (noff + 127) >> 7

      def fl(j, _):
        pltpu.async_copy(cvz.at[j], cblock.at[cidx1.at[j]], sem).wait()
        pltpu.async_copy(cvy.at[j], cblock.at[cidx2.at[j]], sem).wait()
        pltpu.async_copy(cvx.at[j], cblock.at[cidx3.at[j]], sem).wait()
        return jnp.int32(0)

      lax.fori_loop(0, nblk, fl, jnp.int32(0))
      return jnp.int32(0)

    lax.fori_loop(0, BPW // 512, ch, jnp.int32(0))
    plsc.subcore_barrier()
    el0 = sid * CSH_A

    @pl.when(sid < 15)
    def _():
      pltpu.sync_copy(cblock.at[pl.ds(el0, CSH_A)],
                      coors.at[pl.ds(b * CBLK + el0, CSH_A)])

    @pl.when(sid == 15)
    def _():
      pltpu.sync_copy(cblock.at[pl.ds(el0, CSH_B)],
                      coors.at[pl.ds(b * CBLK + el0, CSH_B)])
    plsc.subcore_barrier()

  def scan_queue(b, base, qbase):
    def zr(i):
      count[pl.ds(i * 16, 16)] = zero_i
    _loop(BPW // 16, zr)

    chunk_fire(b, 0, 0)

    def ch(ci, qnb):
      start = ci * ACH
      par = ci & 1
      chunk_wait(b, start, par)

      @pl.when(ci + 1 < NCH)
      def _():
        chunk_fire(b, start + ACH, 1 - par)

      parv = jnp.full((16,), par, jnp.int32)

      def vec(j, off):
        v = linbuf[par, pl.ds(j * 16, 16)]
        m = (v >= lo) & (v < lo + BPW)

        def proc(off):
          locc = jnp.where(m, v - lo, 0)
          cnt = plsc.load_gather(count, [locc], mask=m)
          rc, _lm = plsc.scan_count(locc, m)
          plsc.addupdate_scatter(count, [locc], ones_i, mask=m)
          rank = cnt + rc - 1
          enc = plsc.load_gather(slot, [locc], mask=m)
          slv = (enc >> 1) + base
          valid = m & (rank < MAXP) & (slv < MAXV)
          code = rank * 32768 + slv
          vi = valid.astype(jnp.int32)
          pos = off + plsc.cumsum(vi) - 1
          spos = pos * 6
          rows = j * 16 + iota
          plsc.store_scatter(sq, [spos], code, mask=valid)
          for c in range(5):
            val = plsc.load_gather(pbuf, [parv, rows, cols[c]], mask=valid)
            if c < 4:
              val = (val - jnp.float32(_STARTS[c])) / jnp.float32(_NORMS[c])
            plsc.store_scatter(sq, [spos + 1 + c],
                               plsc.bitcast(val, jnp.int32), mask=valid)
          return off + jnp.sum(vi)

        return lax.cond(jnp.any(m), proc, lambda o: o, off)

      off = lax.fori_loop(0, NVC, vec, jnp.int32(0))
      nblk = (off + 127) >> 7

      def pad(i):
        pp = off + i * 16 + iota
        mk = pp < nblk * 128
        plsc.store_scatter(sq, [pp * 6], jnp.full((16,), SENT, jnp.int32),
                           mask=mk)
      _loop(8, pad)

      def fl(j, _):
        pltpu.sync_copy(sq.at[pl.ds(j * 768, 768)],
                        qarr.at[pl.ds(qbase + (qnb + j) * 768, 768)])
        return jnp.int32(0)

      lax.fori_loop(0, nblk, fl, jnp.int32(0))
      return qnb + nblk

    return lax.fori_loop(0, NCH, ch, jnp.int32(0))

  def quarter_pass(b, qnb, qbase, h):
    qn = qnb * 128
    bcol = b * MAXV
    slo = h * QUART
    nfz = FSH // ZB
    rem = FSH - nfz * ZB

    def zf(i, _):
      pltpu.async_copy(zbuf, fblock.at[pl.ds(sid * FSH + i * ZB, ZB)], sem
                       ).wait()
      return jnp.int32(0)

    lax.fori_loop(0, nfz, zf, jnp.int32(0))
    pltpu.async_copy(zbuf.at[pl.ds(0, rem)],
                     fblock.at[pl.ds(sid * FSH + nfz * ZB, rem)], sem).wait()
    prefill_feat_staging()
    plsc.subcore_barrier()

    nqc = (qn + QCH - 1) // QCH

    def ch(k, _):
      pltpu.sync_copy(qarr.at[pl.ds(qbase + k * (QCH * 6), QCH * 6)],
                      sq.at[pl.ds(0, QCH * 6)])

      def vec(j, off):
        e = j * 16 + iota
        ge = k * QCH + e
        e6 = e * 6
        code = plsc.load_gather(sq, [e6])
        rank = code >> 15
        slv = code & 32767
        sloc = slv - slo
        validq = ((ge < qn) & (rank < MAXP) & (slv < MAXV)
                  & (sloc >= 0) & (sloc < QUART))

        def proc(off):
          d0 = rank * QUART + sloc
          vi = validq.astype(jnp.int32)
          pos = off + plsc.cumsum(vi) - 1
          ph = pos >> 7
          pcl = pos & 127
          for c in range(5):
            dst = d0 + c * (MAXP * QUART)
            plsc.store_scatter(sidx[c], [ph, pcl], dst, mask=validq)
            vv = plsc.load_gather(sq, [e6 + 1 + c], mask=validq)
            plsc.store_scatter(sval[c], [ph, pcl],
                               plsc.bitcast(vv, jnp.float32), mask=validq)
          return off + jnp.sum(vi)

        return lax.cond(jnp.any(validq), proc, lambda o: o, off)

      off = lax.fori_loop(0, QCH // 16, vec, jnp.int32(0))
      nblk = (off + 127) >> 7

      def fl(j, _):
        for c in range(5):
          pltpu.async_copy(sval[c].at[j], fblock.at[sidx[c].at[j]], sem
                           ).wait()
        return jnp.int32(0)

      lax.fori_loop(0, nblk, fl, jnp.int32(0))
      return jnp.int32(0)

    lax.fori_loop(0, nqc, ch, jnp.int32(0))
    plsc.subcore_barrier()
    for kk in range(7):
      p = sid + 16 * kk

      @pl.when((p < NPLANES) & (h < 3))
      def _():
        pltpu.async_copy(fblock.at[pl.ds(p * QUART, QUART)],
                         feat.at[pl.ds(p * PLANE + bcol + slo, QUART)],
                         sem).wait()

      @pl.when((p < NPLANES) & (h == 3))
      def _():
        pltpu.async_copy(fblock.at[pl.ds(p * QUART, QLAST)],
                         feat.at[pl.ds(p * PLANE + bcol + slo, QLAST)],
                         sem).wait()
    plsc.subcore_barrier()

  def batch_body(k, _):
    b = 2 * cid + k
    with jax.named_scope("phase_a"):
      phase_a(b)
    with jax.named_scope("init_cblock"):
      init_cblock(b)
    plsc.subcore_barrier()
    with jax.named_scope("b1_count"):
      b1_count(b)
    with jax.named_scope("prefix"):
      base = prefix_and_base()
    with jax.named_scope("coors_scatter"):
      coors_scatter(b, base)
    qbase = (cid * 16 + sid) * (QSTR * 6)
    with jax.named_scope("scan_queue"):
      qnb = scan_queue(b, base, qbase)

    def qloop(h, _):
      with jax.named_scope("quarter_pass"):
        quarter_pass(b, qnb, qbase, h)
      return jnp.int32(0)

    lax.fori_loop(0, NQ, qloop, jnp.int32(0))
    return jnp.int32(0)

  lax.fori_loop(0, 2, batch_body, jnp.int32(0))


def kernel(points_lst):
  mesh = plsc.VectorSubcoreMesh(core_axis_name="c", subcore_axis_name="s")
  kfn = pl.kernel(
      _sc_body,
      out_type=(
          jax.ShapeDtypeStruct((FEAT,), jnp.float32),
          jax.ShapeDtypeStruct((COLS * 4,), jnp.int32),
          jax.ShapeDtypeStruct((B * NPAD,), jnp.int32),
          jax.ShapeDtypeStruct((32 * QSTR * 6,), jnp.int32),
      ),
      mesh=mesh,
      scratch_types=[
          pltpu.VMEM((2, ACH, C), jnp.float32),
          pltpu.VMEM((2, ACH), jnp.int32),
          pltpu.VMEM((BPW,), jnp.int32),
          pltpu.VMEM((BPW,), jnp.int32),
          pltpu.VMEM((ZB,), jnp.float32),
          pltpu.VMEM((960,), jnp.int32),
          pltpu.VMEM((5, 128), jnp.int32),
          pltpu.VMEM((5, 128), jnp.int32),
          pltpu.VMEM((5, 128), jnp.int32),
          pltpu.VMEM((5, 128), jnp.int32),
          pltpu.VMEM((5, 128), jnp.int32),
          pltpu.VMEM((5, 128), jnp.float32),
          pltpu.VMEM((5, 128), jnp.float32),
          pltpu.VMEM((5, 128), jnp.float32),
          pltpu.VMEM((5, 128), jnp.float32),
          pltpu.VMEM((5, 128), jnp.float32),
          pltpu.VMEM((4, 128), jnp.int32),
          pltpu.VMEM((4, 128), jnp.int32),
          pltpu.VMEM((4, 128), jnp.int32),
          pltpu.VMEM((4, 128), jnp.int32),
          pltpu.VMEM((4, 128), jnp.int32),
          pltpu.VMEM((4, 128), jnp.int32),
          pltpu.VMEM((7680,), jnp.int32),
          pltpu.VMEM_SHARED((FBLK,), jnp.float32),
          pltpu.VMEM_SHARED((CBLK + 64,), jnp.int32),
          pltpu.VMEM_SHARED((16, 16), jnp.int32),
          pltpu.VMEM((16, 16), jnp.int32),
          pltpu.SemaphoreType.DMA,
      ],
      compiler_params=pltpu.CompilerParams(
          needs_layout_passes=False, use_tc_tiling_on_sc=False),
  )
  feat, coors, _lin, _q = kfn(points_lst)
  features = feat.reshape(1, C, MAXP, COLS)
  coors_batch = coors.reshape(COLS, 4)
  return features, coors_batch
